# baseline (speedup 1.0000x reference)
import functools

import jax
import jax.numpy as jnp
from jax import lax
from jax.experimental import pallas as pl
from jax.experimental.pallas import tpu as pltpu
from jax.experimental.pallas import tpu_sc as plsc

N = 10000
E = 320000
H = 128
G = 64
EPS = 1e-5

CHUNK = 128
NCHUNKS = 2560
CPW = NCHUNKS // 32
E_PAD = NCHUNKS * CHUNK
NP_ACC = 10240
BLK = 1000
NBLK = N // BLK

_mesh = plsc.VectorSubcoreMesh(core_axis_name="c", subcore_axis_name="s")


@functools.partial(
    pl.kernel,
    out_type=jax.ShapeDtypeStruct((2, NP_ACC, 16), jnp.float32),
    mesh=_mesh,
    scratch_types=[
        pltpu.VMEM((CPW, CHUNK), jnp.int32),
        pltpu.VMEM((CHUNK, 16), jnp.float32),
        pltpu.VMEM_SHARED((NP_ACC, 16), jnp.float32),
    ],
)
def _sc_deg(dst_hbm, out_hbm, dstv, buf, acc):
    c = lax.axis_index("c")
    s = lax.axis_index("s")
    wid = c * 16 + s

    @pl.loop(0, CHUNK)
    def _zrow(i):
        buf.at[pl.ds(i, 1), pl.ds(0, 16)][...] = jnp.zeros((1, 16), jnp.float32)

    @pl.loop(0, 5)
    def _zacc(k):
        pltpu.sync_copy(buf, acc.at[pl.ds(s * 640 + k * CHUNK, CHUNK)])

    @pl.loop(0, CHUNK)
    def _orow(i):
        buf.at[pl.ds(i, 1), pl.ds(0, 16)][...] = jnp.ones((1, 16), jnp.float32)

    plsc.subcore_barrier()
    pltpu.sync_copy(dst_hbm.at[pl.ds(wid * CPW, CPW)], dstv)

    @pl.loop(0, CPW)
    def _edges(j):
        pltpu.sync_copy(buf, acc.at[dstv.at[j]], add=True)

    plsc.subcore_barrier()
    pltpu.sync_copy(acc.at[pl.ds(s * 640, 640)], out_hbm.at[c, pl.ds(s * 640, 640)])


@functools.partial(
    pl.kernel,
    out_type=jax.ShapeDtypeStruct((2, NP_ACC, H), jnp.float32),
    mesh=_mesh,
    scratch_types=[
        pltpu.VMEM((CPW // 2, CHUNK), jnp.int32),
        pltpu.VMEM((CPW // 2, CHUNK), jnp.int32),
        pltpu.VMEM((CHUNK, H), jnp.float32),
        pltpu.VMEM((CHUNK, H), jnp.float32),
        pltpu.VMEM_SHARED((NP_ACC, H), jnp.float32),
        pltpu.SemaphoreType.DMA,
        pltpu.SemaphoreType.DMA,
    ],
)
def _sc_agg(src_hbm, dst_hbm, g_hbm, out_hbm, srcv, dstv, rows_a, rows_b, acc,
            sem_a, sem_b):
    c = lax.axis_index("c")
    s = lax.axis_index("s")
    wid = c * 16 + s
    hw = CPW // 2

    @pl.loop(0, CHUNK)
    def _zrow(i):
        @pl.loop(0, H, step=16)
        def _zcol(j):
            rows_a.at[pl.ds(i, 1), pl.ds(j, 16)][...] = jnp.zeros((1, 16), jnp.float32)

    @pl.loop(0, 5)
    def _zacc(k):
        pltpu.sync_copy(rows_a, acc.at[pl.ds(s * 640 + k * CHUNK, CHUNK)])

    plsc.subcore_barrier()

    def _gather(j, buf, sem):
        pltpu.make_async_copy(g_hbm.at[srcv.at[j]], buf, sem).start()

    def _gwait(j, buf, sem):
        pltpu.make_async_copy(g_hbm.at[srcv.at[j]], buf, sem).wait()

    @pl.loop(0, 2)
    def _half(h):
        pltpu.sync_copy(src_hbm.at[pl.ds(wid * CPW + h * hw, hw)], srcv)
        pltpu.sync_copy(dst_hbm.at[pl.ds(wid * CPW + h * hw, hw)], dstv)

        @pl.loop(0, hw)
        def _edges(j):
            _gather(j, rows_a, sem_a)
            _gwait(j, rows_a, sem_a)
            pltpu.sync_copy(rows_a, acc.at[dstv.at[j]], add=True)

    plsc.subcore_barrier()
    pltpu.sync_copy(acc.at[pl.ds(s * 640, 640)], out_hbm.at[c, pl.ds(s * 640, 640)])


def _dinv_of(degp_ref):
    cnt = degp_ref[0, :, 0:1] + degp_ref[1, :, 0:1]
    return lax.rsqrt(cnt + 1.0)


def _mm_scale_kernel(degp_ref, x_ref, W_ref, g_ref):
    g_ref[...] = jnp.dot(x_ref[...], W_ref[...],
                         preferred_element_type=jnp.float32) * _dinv_of(degp_ref)


def _comb_kernel(parts_ref, g_ref, degp_ref, b_ref, y_ref, sums_ref):
    i = pl.program_id(0)
    y = (parts_ref[0] + parts_ref[1] + g_ref[...]) * _dinv_of(degp_ref) + b_ref[...]
    y_ref[...] = y

    @pl.when(i == 0)
    def _():
        sums_ref[...] = jnp.zeros_like(sums_ref)

    sums_ref[...] += jnp.stack([jnp.sum(y, axis=0), jnp.sum(y * y, axis=0)])


def _bn_from_sums(sums_ref, gam_ref, bet_ref):
    m = sums_ref[0:1, :] * (1.0 / N)
    v = sums_ref[1:2, :] * (1.0 / N) - m * m
    scale = gam_ref[...] * lax.rsqrt(v + EPS)
    shift = bet_ref[...] - m * scale
    return scale, shift


def _bn_mm_kernel(y_ref, sums_ref, gam_ref, bet_ref, W_ref, degp_ref, g_ref):
    scale, shift = _bn_from_sums(sums_ref, gam_ref, bet_ref)
    h = jnp.maximum(y_ref[...] * scale + shift, 0.0)
    g_ref[...] = jnp.dot(h, W_ref[...],
                         preferred_element_type=jnp.float32) * _dinv_of(degp_ref)


def _pool_kernel(y_ref, sums_ref, gam_ref, bet_ref, batch_ref, s_ref, c_ref):
    i = pl.program_id(0)
    scale, shift = _bn_from_sums(sums_ref, gam_ref, bet_ref)
    h = jnp.maximum(y_ref[...] * scale + shift, 0.0)
    b = batch_ref[0, 0, :]
    onehot = (b[:, None] == lax.broadcasted_iota(jnp.int32, (BLK, G), 1))
    onehot = onehot.astype(jnp.float32)

    @pl.when(i == 0)
    def _():
        s_ref[...] = jnp.zeros_like(s_ref)
        c_ref[...] = jnp.zeros_like(c_ref)

    dn = (((0,), (0,)), ((), ()))
    s_ref[...] += lax.dot_general(onehot, h, dn, preferred_element_type=jnp.float32)
    c_ref[...] += lax.dot_general(onehot, jnp.ones_like(h), dn,
                                  preferred_element_type=jnp.float32)


def _head_kernel(s_ref, c_ref, Wf0_ref, bf0_ref, gamf0_ref, betf0_ref,
                 Wf1_ref, bf1_ref, gamf1_ref, betf1_ref, Wlin_ref, blin_ref,
                 out_ref):
    p = s_ref[...] / jnp.maximum(c_ref[:, 0:1], 1.0)
    for W_r, b_r, g_r, bt_r in ((Wf0_ref, bf0_ref, gamf0_ref, betf0_ref),
                                (Wf1_ref, bf1_ref, gamf1_ref, betf1_ref)):
        p = jnp.dot(p, W_r[...], preferred_element_type=jnp.float32) + b_r[...]
        m = jnp.mean(p, axis=0, keepdims=True)
        v = jnp.mean(p * p, axis=0, keepdims=True) - m * m
        p = jnp.maximum(g_r[...] * (p - m) * lax.rsqrt(v + EPS) + bt_r[...], 0.0)
    out_ref[...] = jnp.maximum(
        jnp.dot(p, Wlin_ref[...], preferred_element_type=jnp.float32) + blin_ref[...],
        0.0)


def _row_spec(blk=BLK):
    return pl.BlockSpec((blk, H), lambda i: (i, 0))


_DEGP_SPEC = pl.BlockSpec((2, BLK, 16), lambda i: (0, i, 0))
_FULL_W = pl.BlockSpec((H, H), lambda i: (0, 0))
_ROWV = pl.BlockSpec((1, H), lambda i: (0, 0))
_SUMS = pl.BlockSpec((2, H), lambda i: (0, 0))


def _mm_scale(degp, x, W):
    return pl.pallas_call(
        _mm_scale_kernel,
        grid=(NBLK,),
        in_specs=[_DEGP_SPEC, _row_spec(), _FULL_W],
        out_specs=_row_spec(),
        out_shape=jax.ShapeDtypeStruct((N, H), jnp.float32),
    )(degp, x, W)


def _comb(parts, g, degp, b):
    return pl.pallas_call(
        _comb_kernel,
        grid=(NBLK,),
        in_specs=[pl.BlockSpec((2, BLK, H), lambda i: (0, i, 0)),
                  _row_spec(), _DEGP_SPEC, _ROWV],
        out_specs=[_row_spec(), _SUMS],
        out_shape=[jax.ShapeDtypeStruct((N, H), jnp.float32),
                   jax.ShapeDtypeStruct((2, H), jnp.float32)],
    )(parts, g, degp, b)


def _bn_mm(y, sums, gam, bet, W, degp):
    return pl.pallas_call(
        _bn_mm_kernel,
        grid=(NBLK,),
        in_specs=[_row_spec(), _SUMS, _ROWV, _ROWV, _FULL_W, _DEGP_SPEC],
        out_specs=_row_spec(),
        out_shape=jax.ShapeDtypeStruct((N, H), jnp.float32),
    )(y, sums, gam, bet, W, degp)


def _pool(y, sums, gam, bet, batch3d):
    return pl.pallas_call(
        _pool_kernel,
        grid=(NBLK,),
        in_specs=[_row_spec(), _SUMS, _ROWV, _ROWV,
                  pl.BlockSpec((1, 1, BLK), lambda i: (i, 0, 0))],
        out_specs=[pl.BlockSpec((G, H), lambda i: (0, 0)),
                   pl.BlockSpec((G, H), lambda i: (0, 0))],
        out_shape=[jax.ShapeDtypeStruct((G, H), jnp.float32),
                   jax.ShapeDtypeStruct((G, H), jnp.float32)],
    )(y, sums, gam, bet, batch3d)


def _head(s, c, Wf0, bf0, gamf0, betf0, Wf1, bf1, gamf1, betf1, Wlin, blin):
    return pl.pallas_call(
        _head_kernel,
        out_shape=jax.ShapeDtypeStruct((G, H), jnp.float32),
    )(s, c, Wf0, bf0, gamf0, betf0, Wf1, bf1, gamf1, betf1, Wlin, blin)


def kernel(x, edge_index, batch, W_g0, b_g0, gam_g0, bet_g0, W_g1, b_g1, gam_g1,
           bet_g1, W_g2, b_g2, gam_g2, bet_g2, W_f0, b_f0, gam_f0, bet_f0,
           W_f1, b_f1, gam_f1, bet_f1, W_lin, b_lin):
    src = edge_index[0]
    dst = edge_index[1]
    pad = E_PAD - E
    srcp = jnp.concatenate([src, jnp.zeros((pad,), jnp.int32)]).reshape(NCHUNKS, CHUNK)
    dummy = N + (jnp.arange(pad, dtype=jnp.int32) % (NP_ACC - N))
    dstp = jnp.concatenate([dst, dummy]).reshape(NCHUNKS, CHUNK)
    batch3d = batch.reshape(NBLK, 1, BLK)
    row = lambda a: a.reshape(1, H)
    Wlin_pad = jnp.pad(W_lin, ((0, 0), (0, H - 1)))
    blin_pad = jnp.pad(b_lin, (0, H - 1)).reshape(1, H)

    degp = _sc_deg(dstp)

    g = _mm_scale(degp, x, W_g0)
    parts = _sc_agg(srcp, dstp, g)
    y, sums = _comb(parts, g, degp, row(b_g0))

    for (W, b, gam, bet) in ((W_g1, b_g1, gam_g0, bet_g0),
                             (W_g2, b_g2, gam_g1, bet_g1)):
        g = _bn_mm(y, sums, row(gam), row(bet), W, degp)
        parts = _sc_agg(srcp, dstp, g)
        y, sums = _comb(parts, g, degp, row(b))

    s, c = _pool(y, sums, row(gam_g2), row(bet_g2), batch3d)
    out = _head(s, c, W_f0, row(b_f0), row(gam_f0), row(bet_f0),
                W_f1, row(b_f1), row(gam_f1), row(bet_f1), Wlin_pad, blin_pad)
    return out[:, 0:1]


# speedup vs baseline: 2.3534x; 2.3534x over previous
import functools

import jax
import jax.numpy as jnp
from jax import lax
from jax.experimental import pallas as pl
from jax.experimental.pallas import tpu as pltpu
from jax.experimental.pallas import tpu_sc as plsc

N = 10000
E = 320000
H = 128
G = 64
EPS = 1e-5

CHUNK = 125
NCHUNKS = E // CHUNK
CPW = NCHUNKS // 32
NP_ACC = 10240
BLK = 1000
NBLK = N // BLK

_mesh = plsc.VectorSubcoreMesh(core_axis_name="c", subcore_axis_name="s")


@functools.partial(
    pl.kernel,
    out_type=jax.ShapeDtypeStruct((2, NP_ACC, 16), jnp.float32),
    mesh=_mesh,
    scratch_types=[
        pltpu.VMEM((CPW, CHUNK), jnp.int32),
        pltpu.VMEM((CHUNK, 16), jnp.float32),
        pltpu.VMEM_SHARED((NP_ACC, 16), jnp.float32),
    ],
)
def _sc_deg(dst_hbm, out_hbm, dstv, buf, acc):
    c = lax.axis_index("c")
    s = lax.axis_index("s")
    wid = c * 16 + s

    @pl.loop(0, CHUNK)
    def _zrow(i):
        buf.at[pl.ds(i, 1), pl.ds(0, 16)][...] = jnp.zeros((1, 16), jnp.float32)

    @pl.loop(0, 5)
    def _zacc(k):
        pltpu.sync_copy(buf, acc.at[pl.ds(s * 640 + k * CHUNK, CHUNK)])

    pltpu.sync_copy(buf.at[pl.ds(0, 15)], acc.at[pl.ds(s * 640 + 625, 15)])

    @pl.loop(0, CHUNK)
    def _orow(i):
        buf.at[pl.ds(i, 1), pl.ds(0, 16)][...] = jnp.ones((1, 16), jnp.float32)

    plsc.subcore_barrier()
    pltpu.sync_copy(dst_hbm.at[pl.ds(wid * CPW, CPW)], dstv)

    @pl.loop(0, CPW)
    def _edges(j):
        pltpu.sync_copy(buf, acc.at[dstv.at[j]], add=True)

    plsc.subcore_barrier()
    pltpu.sync_copy(acc.at[pl.ds(s * 640, 640)], out_hbm.at[c, pl.ds(s * 640, 640)])


@functools.partial(
    pl.kernel,
    out_type=jax.ShapeDtypeStruct((2, NP_ACC, H), jnp.float32),
    mesh=_mesh,
    scratch_types=[
        pltpu.VMEM((CPW // 2, CHUNK), jnp.int32),
        pltpu.VMEM((CPW // 2, CHUNK), jnp.int32),
        pltpu.VMEM((CHUNK, H), jnp.float32),
        pltpu.VMEM((CHUNK, H), jnp.float32),
        pltpu.VMEM_SHARED((NP_ACC, H), jnp.float32),
        pltpu.SemaphoreType.DMA,
        pltpu.SemaphoreType.DMA,
    ],
)
def _sc_agg(src_hbm, dst_hbm, g_hbm, out_hbm, srcv, dstv, rows_a, rows_b, acc,
            sem_a, sem_b):
    c = lax.axis_index("c")
    s = lax.axis_index("s")
    wid = c * 16 + s
    hw = CPW // 2

    @pl.loop(0, CHUNK)
    def _zrow(i):
        @pl.loop(0, H, step=16)
        def _zcol(j):
            rows_a.at[pl.ds(i, 1), pl.ds(j, 16)][...] = jnp.zeros((1, 16), jnp.float32)

    @pl.loop(0, 5)
    def _zacc(k):
        pltpu.sync_copy(rows_a, acc.at[pl.ds(s * 640 + k * CHUNK, CHUNK)])

    pltpu.sync_copy(rows_a.at[pl.ds(0, 15)], acc.at[pl.ds(s * 640 + 625, 15)])
    plsc.subcore_barrier()

    def _gather(j, buf, sem):
        pltpu.make_async_copy(g_hbm.at[srcv.at[j]], buf, sem).start()

    def _gwait(j, buf, sem):
        pltpu.make_async_copy(g_hbm.at[srcv.at[j]], buf, sem).wait()

    @pl.loop(0, 2)
    def _half(h):
        pltpu.sync_copy(src_hbm.at[pl.ds(wid * CPW + h * hw, hw)], srcv)
        pltpu.sync_copy(dst_hbm.at[pl.ds(wid * CPW + h * hw, hw)], dstv)

        @pl.loop(0, hw)
        def _edges(j):
            _gather(j, rows_a, sem_a)
            _gwait(j, rows_a, sem_a)
            pltpu.sync_copy(rows_a, acc.at[dstv.at[j]], add=True)

    plsc.subcore_barrier()
    pltpu.sync_copy(acc.at[pl.ds(s * 640, 640)], out_hbm.at[c, pl.ds(s * 640, 640)])


def _dinv_of(degp_ref):
    cnt = degp_ref[0, :, 0:1] + degp_ref[1, :, 0:1]
    return lax.rsqrt(cnt + 1.0)


def _mm_scale_kernel(degp_ref, x_ref, W_ref, g_ref):
    g_ref[...] = jnp.dot(x_ref[...], W_ref[...],
                         preferred_element_type=jnp.float32) * _dinv_of(degp_ref)


def _comb_kernel(parts_ref, g_ref, degp_ref, b_ref, y_ref, sums_ref):
    i = pl.program_id(0)
    y = (parts_ref[0] + parts_ref[1] + g_ref[...]) * _dinv_of(degp_ref) + b_ref[...]
    y_ref[...] = y

    @pl.when(i == 0)
    def _():
        sums_ref[...] = jnp.zeros_like(sums_ref)

    sums_ref[...] += jnp.stack([jnp.sum(y, axis=0), jnp.sum(y * y, axis=0)])


def _bn_from_sums(sums_ref, gam_ref, bet_ref):
    m = sums_ref[0:1, :] * (1.0 / N)
    v = sums_ref[1:2, :] * (1.0 / N) - m * m
    scale = gam_ref[...] * lax.rsqrt(v + EPS)
    shift = bet_ref[...] - m * scale
    return scale, shift


def _bn_mm_kernel(y_ref, sums_ref, gam_ref, bet_ref, W_ref, degp_ref, g_ref):
    scale, shift = _bn_from_sums(sums_ref, gam_ref, bet_ref)
    h = jnp.maximum(y_ref[...] * scale + shift, 0.0)
    g_ref[...] = jnp.dot(h, W_ref[...],
                         preferred_element_type=jnp.float32) * _dinv_of(degp_ref)


def _pool_kernel(y_ref, sums_ref, gam_ref, bet_ref, batch_ref, s_ref, c_ref):
    i = pl.program_id(0)
    scale, shift = _bn_from_sums(sums_ref, gam_ref, bet_ref)
    h = jnp.maximum(y_ref[...] * scale + shift, 0.0)
    b = batch_ref[0, 0, :]
    onehot = (b[:, None] == lax.broadcasted_iota(jnp.int32, (BLK, G), 1))
    onehot = onehot.astype(jnp.float32)

    @pl.when(i == 0)
    def _():
        s_ref[...] = jnp.zeros_like(s_ref)
        c_ref[...] = jnp.zeros_like(c_ref)

    dn = (((0,), (0,)), ((), ()))
    s_ref[...] += lax.dot_general(onehot, h, dn, preferred_element_type=jnp.float32)
    c_ref[...] += lax.dot_general(onehot, jnp.ones_like(h), dn,
                                  preferred_element_type=jnp.float32)


def _head_kernel(s_ref, c_ref, Wf0_ref, bf0_ref, gamf0_ref, betf0_ref,
                 Wf1_ref, bf1_ref, gamf1_ref, betf1_ref, Wlin_ref, blin_ref,
                 out_ref):
    p = s_ref[...] / jnp.maximum(c_ref[:, 0:1], 1.0)
    for W_r, b_r, g_r, bt_r in ((Wf0_ref, bf0_ref, gamf0_ref, betf0_ref),
                                (Wf1_ref, bf1_ref, gamf1_ref, betf1_ref)):
        p = jnp.dot(p, W_r[...], preferred_element_type=jnp.float32) + b_r[...]
        m = jnp.mean(p, axis=0, keepdims=True)
        v = jnp.mean(p * p, axis=0, keepdims=True) - m * m
        p = jnp.maximum(g_r[...] * (p - m) * lax.rsqrt(v + EPS) + bt_r[...], 0.0)
    out_ref[...] = jnp.maximum(
        jnp.dot(p, Wlin_ref[...], preferred_element_type=jnp.float32) + blin_ref[...],
        0.0)


def _row_spec(blk=BLK):
    return pl.BlockSpec((blk, H), lambda i: (i, 0))


_DEGP_SPEC = pl.BlockSpec((2, BLK, 16), lambda i: (0, i, 0))
_FULL_W = pl.BlockSpec((H, H), lambda i: (0, 0))
_ROWV = pl.BlockSpec((1, H), lambda i: (0, 0))
_SUMS = pl.BlockSpec((2, H), lambda i: (0, 0))


def _mm_scale(degp, x, W):
    return pl.pallas_call(
        _mm_scale_kernel,
        grid=(NBLK,),
        in_specs=[_DEGP_SPEC, _row_spec(), _FULL_W],
        out_specs=_row_spec(),
        out_shape=jax.ShapeDtypeStruct((N, H), jnp.float32),
    )(degp, x, W)


def _comb(parts, g, degp, b):
    return pl.pallas_call(
        _comb_kernel,
        grid=(NBLK,),
        in_specs=[pl.BlockSpec((2, BLK, H), lambda i: (0, i, 0)),
                  _row_spec(), _DEGP_SPEC, _ROWV],
        out_specs=[_row_spec(), _SUMS],
        out_shape=[jax.ShapeDtypeStruct((N, H), jnp.float32),
                   jax.ShapeDtypeStruct((2, H), jnp.float32)],
    )(parts, g, degp, b)


def _bn_mm(y, sums, gam, bet, W, degp):
    return pl.pallas_call(
        _bn_mm_kernel,
        grid=(NBLK,),
        in_specs=[_row_spec(), _SUMS, _ROWV, _ROWV, _FULL_W, _DEGP_SPEC],
        out_specs=_row_spec(),
        out_shape=jax.ShapeDtypeStruct((N, H), jnp.float32),
    )(y, sums, gam, bet, W, degp)


def _pool(y, sums, gam, bet, batch3d):
    return pl.pallas_call(
        _pool_kernel,
        grid=(NBLK,),
        in_specs=[_row_spec(), _SUMS, _ROWV, _ROWV,
                  pl.BlockSpec((1, 1, BLK), lambda i: (i, 0, 0))],
        out_specs=[pl.BlockSpec((G, H), lambda i: (0, 0)),
                   pl.BlockSpec((G, H), lambda i: (0, 0))],
        out_shape=[jax.ShapeDtypeStruct((G, H), jnp.float32),
                   jax.ShapeDtypeStruct((G, H), jnp.float32)],
    )(y, sums, gam, bet, batch3d)


def _head(s, c, Wf0, bf0, gamf0, betf0, Wf1, bf1, gamf1, betf1, Wlin, blin):
    return pl.pallas_call(
        _head_kernel,
        out_shape=jax.ShapeDtypeStruct((G, H), jnp.float32),
    )(s, c, Wf0, bf0, gamf0, betf0, Wf1, bf1, gamf1, betf1, Wlin, blin)


def kernel(x, edge_index, batch, W_g0, b_g0, gam_g0, bet_g0, W_g1, b_g1, gam_g1,
           bet_g1, W_g2, b_g2, gam_g2, bet_g2, W_f0, b_f0, gam_f0, bet_f0,
           W_f1, b_f1, gam_f1, bet_f1, W_lin, b_lin):
    srcp = edge_index[0].reshape(NCHUNKS, CHUNK)
    dstp = edge_index[1].reshape(NCHUNKS, CHUNK)
    batch3d = batch.reshape(NBLK, 1, BLK)
    row = lambda a: a.reshape(1, H)
    Wlin_pad = jnp.pad(W_lin, ((0, 0), (0, H - 1)))
    blin_pad = jnp.pad(b_lin, (0, H - 1)).reshape(1, H)

    degp = _sc_deg(dstp)

    g = _mm_scale(degp, x, W_g0)
    parts = _sc_agg(srcp, dstp, g)
    y, sums = _comb(parts, g, degp, row(b_g0))

    for (W, b, gam, bet) in ((W_g1, b_g1, gam_g0, bet_g0),
                             (W_g2, b_g2, gam_g1, bet_g1)):
        g = _bn_mm(y, sums, row(gam), row(bet), W, degp)
        parts = _sc_agg(srcp, dstp, g)
        y, sums = _comb(parts, g, degp, row(b))

    s, c = _pool(y, sums, row(gam_g2), row(bet_g2), batch3d)
    out = _head(s, c, W_f0, row(b_f0), row(gam_f0), row(bet_f0),
                W_f1, row(b_f1), row(gam_f1), row(bet_f1), Wlin_pad, blin_pad)
    return out[:, 0:1]
